# baseline (speedup 1.0000x reference)
import functools

import jax
import jax.numpy as jnp
from jax import lax
from jax.experimental import pallas as pl
from jax.experimental.pallas import tpu as pltpu

_TAPS = {(0, -1): (0,), (0, 0): (1, 2), (1, 0): (0, 1), (1, 1): (2,)}
_SHIFTS = ((-1, 0), (0, 1))


def _upconv_kernel(x_ref, w_ref, b_ref, o_ref, *, width, height):
    xb = x_ref[0].astype(jnp.bfloat16)
    L, cin = xb.shape
    cout = w_ref.shape[2]

    row = lax.broadcasted_iota(jnp.int32, (L, cin), 0)
    colv = row % width
    zero = jnp.zeros_like(xb)

    zpad = jnp.zeros((width, cin), dtype=xb.dtype)
    vpads = {}
    for b in (-1, 0, 1):
        if b == -1:
            v = jnp.where(colv == 0, zero, pltpu.roll(xb, 1, 0))
        elif b == 0:
            v = xb
        else:
            v = jnp.where(colv == width - 1, zero, pltpu.roll(xb, L - 1, 0))
        vpads[b] = jnp.concatenate([zpad, v, zpad], axis=0)

    bias = b_ref[...]
    idx = 0
    for py in (0, 1):
        for px in (0, 1):
            acc = None
            for a in _SHIFTS[py]:
                for b in _SHIFTS[px]:
                    lhs = vpads[b][width * (1 + a):width * (1 + a) + L, :]
                    contrib = jnp.dot(lhs, w_ref[idx],
                                      preferred_element_type=jnp.float32)
                    acc = contrib if acc is None else acc + contrib
                    idx += 1
            acc = jnp.maximum(acc + bias, 0.0).astype(o_ref.dtype)
            lanes = min(128, cout)
            ch = cout // lanes
            for h in range(ch):
                piece = acc[:, lanes * h:lanes * (h + 1)]
                o_ref[0, pl.dslice(py, height, 2),
                      pl.dslice(ch * px + h, width, 2 * ch), :] = (
                    piece.reshape(height, width, lanes))


def _fold_weights(weight_oihw):
    blocks = []
    for py in (0, 1):
        for px in (0, 1):
            for a in _SHIFTS[py]:
                for b in _SHIFTS[px]:
                    w_sum = None
                    for kh in _TAPS[(py, a)]:
                        for kw in _TAPS[(px, b)]:
                            t = weight_oihw[:, :, kh, kw]
                            w_sum = t if w_sum is None else w_sum + t
                    blocks.append(w_sum.T)
    return jnp.stack(blocks, axis=0)


def kernel(x_nchw, weight_oihw, bias):
    N, Cin, H, W = x_nchw.shape
    Cout = weight_oihw.shape[0]
    out_dtype = x_nchw.dtype
    L = H * W

    w16 = _fold_weights(weight_oihw).astype(jnp.bfloat16)
    b2d = bias.reshape(1, Cout).astype(jnp.float32)
    x_rows = jnp.transpose(x_nchw, (0, 2, 3, 1)).reshape(N, L, Cin)

    _kfn = functools.partial(_upconv_kernel, width=W, height=H)

    lanes = min(128, Cout)
    ch = Cout // lanes
    out_k = pl.pallas_call(
        _kfn,
        out_shape=jax.ShapeDtypeStruct((N, 2 * H, 2 * W * ch, lanes),
                                       out_dtype),
        grid=(N,),
        in_specs=[
            pl.BlockSpec((1, L, Cin), lambda n: (n, 0, 0)),
            pl.BlockSpec((16, Cin, Cout), lambda n: (0, 0, 0)),
            pl.BlockSpec((1, Cout), lambda n: (0, 0)),
        ],
        out_specs=pl.BlockSpec((1, 2 * H, 2 * W * ch, lanes),
                               lambda n: (n, 0, 0, 0)),
        compiler_params=pltpu.CompilerParams(
            dimension_semantics=("parallel",),
            vmem_limit_bytes=60 * 1024 * 1024),
    )(x_rows, w16, b2d)

    out = out_k.reshape(N, 2 * H, 2 * W, Cout)
    return jnp.transpose(out, (0, 3, 1, 2))


# speedup vs baseline: 2.2530x; 2.2530x over previous
import functools

import jax
import jax.numpy as jnp
from jax import lax
from jax.experimental import pallas as pl
from jax.experimental.pallas import tpu as pltpu

_TAPS = {(0, -1): (0,), (0, 0): (1, 2), (1, 0): (0, 1), (1, 1): (2,)}
_SHIFTS = ((-1, 0), (0, 1))


def _upconv_kernel(x_ref, w_ref, b_ref, o_ref, *, width, height):
    xb = x_ref[0].astype(jnp.bfloat16)
    L, cin = xb.shape
    cout = w_ref.shape[2]

    row = lax.broadcasted_iota(jnp.int32, (L, cin), 0)
    colv = row % width
    zero = jnp.zeros_like(xb)

    zpad = jnp.zeros((width, cin), dtype=xb.dtype)
    vpads = {}
    for b in (-1, 0, 1):
        if b == -1:
            v = jnp.where(colv == 0, zero, pltpu.roll(xb, 1, 0))
        elif b == 0:
            v = xb
        else:
            v = jnp.where(colv == width - 1, zero, pltpu.roll(xb, L - 1, 0))
        vpads[b] = jnp.concatenate([zpad, v, zpad], axis=0)

    bias = b_ref[...]
    idx = 0
    for py in (0, 1):
        for px in (0, 1):
            acc = None
            for a in _SHIFTS[py]:
                for b in _SHIFTS[px]:
                    lhs = vpads[b][width * (1 + a):width * (1 + a) + L, :]
                    contrib = jnp.dot(lhs, w_ref[idx],
                                      preferred_element_type=jnp.float32)
                    acc = contrib if acc is None else acc + contrib
                    idx += 1
            acc = jnp.maximum(acc + bias, 0.0).astype(o_ref.dtype)
            lanes = min(128, cout)
            ch = cout // lanes
            for t in range(ch):
                piece = acc[:, lanes * t:lanes * (t + 1)]
                o_ref[0, pl.dslice(py, height, 2), :, t,
                      pl.dslice(px, 4, 2), :] = (
                    piece.reshape(height, width // 4, 4, lanes))


def _fold_weights(weight_oihw):
    blocks = []
    for py in (0, 1):
        for px in (0, 1):
            for a in _SHIFTS[py]:
                for b in _SHIFTS[px]:
                    w_sum = None
                    for kh in _TAPS[(py, a)]:
                        for kw in _TAPS[(px, b)]:
                            t = weight_oihw[:, :, kh, kw]
                            w_sum = t if w_sum is None else w_sum + t
                    blocks.append(w_sum.T)
    return jnp.stack(blocks, axis=0)


def kernel(x_nchw, weight_oihw, bias):
    N, Cin, H, W = x_nchw.shape
    Cout = weight_oihw.shape[0]
    out_dtype = x_nchw.dtype
    L = H * W

    w16 = _fold_weights(weight_oihw).astype(jnp.bfloat16)
    b2d = bias.reshape(1, Cout).astype(jnp.float32)
    x_rows = jnp.transpose(x_nchw, (0, 2, 3, 1)).reshape(N, L, Cin)

    _kfn = functools.partial(_upconv_kernel, width=W, height=H)

    lanes = min(128, Cout)
    ch = Cout // lanes
    oshape = (N, 2 * H, 2 * W // 8, ch, 8, lanes)
    out_k = pl.pallas_call(
        _kfn,
        out_shape=jax.ShapeDtypeStruct(oshape, out_dtype),
        grid=(N,),
        in_specs=[
            pl.BlockSpec((1, L, Cin), lambda n: (n, 0, 0)),
            pl.BlockSpec((16, Cin, Cout), lambda n: (0, 0, 0)),
            pl.BlockSpec((1, Cout), lambda n: (0, 0)),
        ],
        out_specs=pl.BlockSpec((1,) + oshape[1:],
                               lambda n: (n, 0, 0, 0, 0, 0)),
        compiler_params=pltpu.CompilerParams(
            dimension_semantics=("parallel",),
            vmem_limit_bytes=60 * 1024 * 1024),
    )(x_rows, w16, b2d)

    out = jnp.transpose(out_k, (0, 3, 5, 1, 2, 4))
    return out.reshape(N, Cout, 2 * H, 2 * W)
